# baseline (speedup 1.0000x reference)
import functools

import jax
import jax.numpy as jnp
from jax import lax
from jax.experimental import pallas as pl
from jax.experimental.pallas import tpu as pltpu
from jax.experimental.pallas import tpu_sc as plsc

T = 2048
D = 768
F = 2048
NE = 64
K = 2
TM = 128
MAX_TILES = (T * K) // TM + NE
R = MAX_TILES * TM

NC = 2
NS = 16
NW = NC * NS


def _router_body(x_ref, wg_ref, e1_ref, e2_ref, w1_ref, w2_ref, cnt_ref,
                 aux_ref):
    x = x_ref[...]
    logits = jnp.dot(x, wg_ref[...], preferred_element_type=jnp.float32)
    m = jnp.max(logits, axis=-1, keepdims=True)
    p = jnp.exp(logits - m)
    p = p / jnp.sum(p, axis=-1, keepdims=True)
    iota = lax.broadcasted_iota(jnp.int32, (T, NE), 1)
    p1 = jnp.max(p, axis=-1, keepdims=True)
    e1 = jnp.min(jnp.where(p == p1, iota, NE), axis=-1, keepdims=True)
    pm = jnp.where(iota == e1, -1e30, p)
    p2 = jnp.max(pm, axis=-1, keepdims=True)
    e2 = jnp.min(jnp.where(pm == p2, iota, NE), axis=-1, keepdims=True)
    s = p1 + p2
    e1_ref[...] = e1
    e2_ref[...] = e2
    w1_ref[...] = p1 / s
    w2_ref[...] = p2 / s
    hit1 = (iota == e1).astype(jnp.float32)
    hit2 = (iota == e2).astype(jnp.float32)
    cnt = jnp.sum(hit1 + hit2, axis=0)
    cnt_ref[...] = cnt
    pmean = jnp.mean(p, axis=0)
    f = cnt / jnp.sum(cnt)
    aux_ref[...] = (NE * jnp.sum(f * pmean))[None, None]


def _run_router(x2d, wg):
    return pl.pallas_call(
        _router_body,
        out_shape=(
            jax.ShapeDtypeStruct((T, 1), jnp.int32),
            jax.ShapeDtypeStruct((T, 1), jnp.int32),
            jax.ShapeDtypeStruct((T, 1), jnp.float32),
            jax.ShapeDtypeStruct((T, 1), jnp.float32),
            jax.ShapeDtypeStruct((NE,), jnp.float32),
            jax.ShapeDtypeStruct((1, 1), jnp.float32),
        ),
    )(x2d, wg)


_G_CHUNK = 64
_G_PER_W = R // NW
_G_ITERS = _G_PER_W // _G_CHUNK

@functools.lru_cache(maxsize=None)
def _sc_kernels():
    mesh = plsc.VectorSubcoreMesh(core_axis_name="c", subcore_axis_name="s")

    @functools.partial(
        pl.kernel,
        out_type=jax.ShapeDtypeStruct((R, D), jnp.float32),
        mesh=mesh,
        scratch_types=[
            pltpu.VMEM((_G_ITERS, _G_CHUNK), jnp.int32),
            pltpu.VMEM((_G_CHUNK, D), jnp.float32),
            pltpu.VMEM((_G_CHUNK, D), jnp.float32),
            pltpu.SemaphoreType.DMA,
            pltpu.SemaphoreType.DMA,
            pltpu.SemaphoreType.DMA,
            pltpu.SemaphoreType.DMA,
        ],
    )
    def _sc_gather(x_hbm, idx_hbm, out_hbm, idx_v, b0, b1, si0, si1, so0,
                   so1):
        wid = lax.axis_index("s") * NC + lax.axis_index("c")
        base = wid * _G_PER_W
        pltpu.sync_copy(idx_hbm.at[wid], idx_v)
        bufs = (b0, b1)
        sin = (si0, si1)
        sout = (so0, so1)
        cp_in = [None, None]
        cp_out = [None, None]
        cp_in[0] = pltpu.async_copy(x_hbm.at[idx_v.at[0]], bufs[0], sin[0])
        for j in range(_G_ITERS):
            k = j % 2
            nk = (j + 1) % 2
            if j + 1 < _G_ITERS:
                if cp_out[nk] is not None:
                    cp_out[nk].wait()
                cp_in[nk] = pltpu.async_copy(
                    x_hbm.at[idx_v.at[j + 1]], bufs[nk], sin[nk])
            cp_in[k].wait()
            cp_out[k] = pltpu.async_copy(
                bufs[k], out_hbm.at[pl.ds(base + j * _G_CHUNK, _G_CHUNK)],
                sout[k])
        cp_out[0].wait()
        cp_out[1].wait()

    @functools.partial(
        pl.kernel,
        out_type=jax.ShapeDtypeStruct((T, D), jnp.float32),
        mesh=mesh,
        scratch_types=[
            pltpu.VMEM((_C_PER_W,), jnp.int32),
            pltpu.VMEM((_C_PER_W,), jnp.int32),
            pltpu.VMEM((_C_PER_W, D), jnp.float32),
            pltpu.VMEM((_C_PER_W, D), jnp.float32),
            pltpu.SemaphoreType.DMA,
        ],
    )
    def _sc_combine(ys_hbm, pos0_hbm, pos1_hbm, out_hbm, i0_v, i1_v, b0_v,
                    b1_v, sem):
        wid = lax.axis_index("s") * NC + lax.axis_index("c")
        base = wid * _C_PER_W
        pltpu.sync_copy(pos0_hbm.at[pl.ds(base, _C_PER_W)], i0_v)
        pltpu.sync_copy(pos1_hbm.at[pl.ds(base, _C_PER_W)], i1_v)
        pltpu.async_copy(ys_hbm.at[i0_v], b0_v, sem).wait()
        pltpu.async_copy(ys_hbm.at[i1_v], b1_v, sem).wait()

        def _row(r, carry):
            for c in range(_C_VCH):
                sl = pl.ds(c * 16, 16)
                b0_v[r, sl] = b0_v[r, sl] + b1_v[r, sl]
            return carry

        lax.fori_loop(0, _C_PER_W, _row, 0)
        pltpu.sync_copy(b0_v, out_hbm.at[pl.ds(base, _C_PER_W)])

    return _sc_gather, _sc_combine


NF = 4
FB = F // NF


def _ffn_body(eid_ref, val_ref, x_ref, wg_ref, wu_ref, wd_ref, tok_ref,
              w_ref, out_ref, xt_ref, acc_ref):
    i = pl.program_id(0)
    fj = pl.program_id(1)

    @pl.when(val_ref[i] == 1)
    def _compute():
        @pl.when(fj == 0)
        def _dispatch():
            tok = tok_ref[0]
            iot = lax.broadcasted_iota(jnp.int32, (TM, T), 1)
            sel = (iot == tok).astype(jnp.float32)
            xt_ref[...] = jnp.dot(sel, x_ref[...],
                                  preferred_element_type=jnp.float32)

        xt = xt_ref[...]
        g = jnp.dot(xt, wg_ref[0], preferred_element_type=jnp.float32)
        u = jnp.dot(xt, wu_ref[0], preferred_element_type=jnp.float32)
        h = g * jax.nn.sigmoid(g) * u
        c = jnp.dot(h, wd_ref[0], preferred_element_type=jnp.float32)

        @pl.when(fj == 0)
        def _init():
            acc_ref[...] = c

        @pl.when(fj > 0)
        def _acc():
            acc_ref[...] += c

        @pl.when(fj == NF - 1)
        def _emit():
            out_ref[...] = acc_ref[...] * w_ref[0]


def _run_ffn(tile_eid, tile_valid, x2d, w_gate, w_up, w_down, tok_pad3,
             w_pad3):
    grid_spec = pltpu.PrefetchScalarGridSpec(
        num_scalar_prefetch=2,
        grid=(MAX_TILES, NF),
        in_specs=[
            pl.BlockSpec((T, D), lambda i, fj, eid, val: (0, 0)),
            pl.BlockSpec((1, D, FB), lambda i, fj, eid, val: (eid[i], 0, fj)),
            pl.BlockSpec((1, D, FB), lambda i, fj, eid, val: (eid[i], 0, fj)),
            pl.BlockSpec((1, FB, D), lambda i, fj, eid, val: (eid[i], fj, 0)),
            pl.BlockSpec((1, TM, 1), lambda i, fj, eid, val: (i, 0, 0)),
            pl.BlockSpec((1, TM, 1), lambda i, fj, eid, val: (i, 0, 0)),
        ],
        out_specs=pl.BlockSpec((TM, D), lambda i, fj, eid, val: (i, 0)),
        scratch_shapes=[
            pltpu.VMEM((TM, D), jnp.float32),
            pltpu.VMEM((TM, D), jnp.float32),
        ],
    )
    return pl.pallas_call(
        _ffn_body,
        grid_spec=grid_spec,
        out_shape=jax.ShapeDtypeStruct((R, D), jnp.float32),
    )(tile_eid, tile_valid, x2d, w_gate, w_up, w_down, tok_pad3, w_pad3)


_C_PER_W = T // NW
_C_VCH = D // 16


def kernel(x, Wg, W_gate, W_up, W_down):
    x2d = x.reshape(T, D)
    e1, e2, w1, w2, cnt_f, aux = _run_router(x2d, Wg)

    e_all = jnp.concatenate([e1, e2], axis=1).reshape(-1)
    w_all = jnp.concatenate([w1, w2], axis=1).reshape(-1)
    order = jnp.argsort(e_all, stable=True)
    sorted_e = e_all[order]
    counts = cnt_f.astype(jnp.int32)
    tiles_pe = (counts + TM - 1) // TM
    tile_cum = jnp.cumsum(tiles_pe)
    total_tiles = tile_cum[NE - 1]
    padded_start = (tile_cum - tiles_pe) * TM
    grp_start = jnp.cumsum(counts) - counts
    i_all = jnp.arange(T * K, dtype=jnp.int32)
    pos_sorted = padded_start[sorted_e] + (i_all - grp_start[sorted_e])
    tok_pad = jnp.zeros((R,), jnp.int32).at[pos_sorted].set(
        (order // K).astype(jnp.int32))
    w_pad = jnp.zeros((R,), jnp.float32).at[pos_sorted].set(w_all[order])
    pos_flat = jnp.zeros((T * K,), jnp.int32).at[order].set(pos_sorted)
    pos0 = pos_flat[0::K]
    pos1 = pos_flat[1::K]
    t_iota = jnp.arange(MAX_TILES, dtype=jnp.int32)
    tile_eid = jnp.minimum(
        jnp.searchsorted(tile_cum, t_iota, side="right"), NE - 1
    ).astype(jnp.int32)
    tile_valid = (t_iota < total_tiles).astype(jnp.int32)

    sc_gather, sc_combine = _sc_kernels()
    tok_pad3 = tok_pad.reshape(MAX_TILES, TM, 1)
    w_pad3 = w_pad.reshape(MAX_TILES, TM, 1)
    ys = _run_ffn(tile_eid, tile_valid, x2d, W_gate, W_up, W_down, tok_pad3,
                  w_pad3)
    out2d = sc_combine(ys, pos0, pos1)

    return out2d.reshape(1, T, D), aux[0, 0], cnt_f


# speedup vs baseline: 2.0594x; 2.0594x over previous
import functools

import jax
import jax.numpy as jnp
from jax import lax
from jax.experimental import pallas as pl
from jax.experimental.pallas import tpu as pltpu
from jax.experimental.pallas import tpu_sc as plsc

T = 2048
D = 768
F = 2048
NE = 64
K = 2
TM = 128
MAX_TILES = (T * K) // TM + NE
R = MAX_TILES * TM

NC = 2
NS = 16
NW = NC * NS


def _router_body(x_ref, wg_ref, cnt_ref, aux_ref, w1_ref, w2_ref, p1_ref,
                 p2_ref, eid_ref, val_ref):
    x = x_ref[...]
    logits = jnp.dot(x, wg_ref[...], preferred_element_type=jnp.float32)
    m = jnp.max(logits, axis=-1, keepdims=True)
    p = jnp.exp(logits - m)
    p = p / jnp.sum(p, axis=-1, keepdims=True)
    iota = lax.broadcasted_iota(jnp.int32, (T, NE), 1)
    p1 = jnp.max(p, axis=-1, keepdims=True)
    e1 = jnp.min(jnp.where(p == p1, iota, NE), axis=-1, keepdims=True)
    pm = jnp.where(iota == e1, -1e30, p)
    p2 = jnp.max(pm, axis=-1, keepdims=True)
    e2 = jnp.min(jnp.where(pm == p2, iota, NE), axis=-1, keepdims=True)
    s = p1 + p2
    w1_ref[...] = p1 / s
    w2_ref[...] = p2 / s
    hit1 = (iota == e1).astype(jnp.float32)
    hit2 = (iota == e2).astype(jnp.float32)
    hits = hit1 + hit2
    cnt = jnp.sum(hits, axis=0)
    cnt_ref[...] = cnt
    pmean = jnp.mean(p, axis=0)
    f = cnt / jnp.sum(cnt)
    aux_ref[...] = (NE * jnp.sum(f * pmean))[None, None]

    cnt_row = cnt[None, :]
    tiles_pe = (cnt_row.astype(jnp.int32) + TM - 1) // TM
    i64r = lax.broadcasted_iota(jnp.int32, (NE, NE), 0)
    i64c = lax.broadcasted_iota(jnp.int32, (NE, NE), 1)
    m_le = (i64r <= i64c).astype(jnp.float32)
    tile_cum = jnp.dot(tiles_pe.astype(jnp.float32), m_le,
                       preferred_element_type=jnp.float32).astype(jnp.int32)
    padded_start = ((tile_cum - tiles_pe) * TM).astype(jnp.float32)

    tb = 256
    ibr = lax.broadcasted_iota(jnp.int32, (tb, tb), 0)
    ibc = lax.broadcasted_iota(jnp.int32, (tb, tb), 1)
    l_strict = (ibc < ibr).astype(jnp.float32)
    off = jnp.zeros((1, NE), jnp.float32)
    for b in range(T // tb):
        sl = slice(b * tb, (b + 1) * tb)
        hb = hits[sl]
        cumex_b = jnp.dot(l_strict, hb,
                          preferred_element_type=jnp.float32) + off
        off = off + jnp.sum(hb, axis=0, keepdims=True)
        tgt_b = padded_start + cumex_b
        p1_ref[sl, :] = jnp.sum(hit1[sl] * tgt_b, axis=-1,
                                keepdims=True).astype(jnp.int32)
        p2_ref[sl, :] = jnp.sum(hit2[sl] * tgt_b, axis=-1,
                                keepdims=True).astype(jnp.int32)

    ti = lax.broadcasted_iota(jnp.int32, (MAX_TILES, NE), 0)
    eid = jnp.sum((tile_cum <= ti).astype(jnp.int32), axis=-1,
                  keepdims=True)
    eid_ref[...] = jnp.minimum(eid, NE - 1)
    total = jnp.max(tile_cum)
    vi = lax.broadcasted_iota(jnp.int32, (MAX_TILES, 1), 0)
    val_ref[...] = (vi < total).astype(jnp.int32)


def _run_router(x2d, wg):
    return pl.pallas_call(
        _router_body,
        out_shape=(
            jax.ShapeDtypeStruct((NE,), jnp.float32),
            jax.ShapeDtypeStruct((1, 1), jnp.float32),
            jax.ShapeDtypeStruct((T, 1), jnp.float32),
            jax.ShapeDtypeStruct((T, 1), jnp.float32),
            jax.ShapeDtypeStruct((T, 1), jnp.int32),
            jax.ShapeDtypeStruct((T, 1), jnp.int32),
            jax.ShapeDtypeStruct((MAX_TILES, 1), jnp.int32),
            jax.ShapeDtypeStruct((MAX_TILES, 1), jnp.int32),
        ),
    )(x2d, wg)


@functools.lru_cache(maxsize=None)
def _sc_kernels():
    mesh = plsc.VectorSubcoreMesh(core_axis_name="c", subcore_axis_name="s")

    @functools.partial(
        pl.kernel,
        out_type=jax.ShapeDtypeStruct((T, D), jnp.float32),
        mesh=mesh,
        scratch_types=[
            pltpu.VMEM((_C_PER_W,), jnp.int32),
            pltpu.VMEM((_C_PER_W,), jnp.int32),
            pltpu.VMEM((_C_PER_W, D), jnp.float32),
            pltpu.VMEM((_C_PER_W, D), jnp.float32),
            pltpu.SemaphoreType.DMA,
        ],
    )
    def _sc_combine(ys_hbm, pos0_hbm, pos1_hbm, out_hbm, i0_v, i1_v, b0_v,
                    b1_v, sem):
        wid = lax.axis_index("s") * NC + lax.axis_index("c")
        base = wid * _C_PER_W
        pltpu.sync_copy(pos0_hbm.at[pl.ds(base, _C_PER_W)], i0_v)
        pltpu.sync_copy(pos1_hbm.at[pl.ds(base, _C_PER_W)], i1_v)
        pltpu.async_copy(ys_hbm.at[i0_v], b0_v, sem).wait()
        pltpu.async_copy(ys_hbm.at[i1_v], b1_v, sem).wait()

        def _row(r, carry):
            for c in range(_C_VCH):
                sl = pl.ds(c * 16, 16)
                b0_v[r, sl] = b0_v[r, sl] + b1_v[r, sl]
            return carry

        lax.fori_loop(0, _C_PER_W, _row, 0)
        pltpu.sync_copy(b0_v, out_hbm.at[pl.ds(base, _C_PER_W)])

    return _sc_combine


def _ffn_body(eid_ref, val_ref, x_ref, wg_ref, wu_ref, wd_ref, pos0_ref,
              pos1_ref, w1_ref, w2_ref, out_ref):
    i = pl.program_id(0)

    @pl.when(val_ref[i] == 1)
    def _compute():
        rid = lax.broadcasted_iota(jnp.int32, (TM, T), 0) + i * TM
        m0 = pos0_ref[...] == rid
        m1 = pos1_ref[...] == rid
        sel = (m0 | m1).astype(jnp.float32)
        wvec = jnp.sum(jnp.where(m0, w1_ref[...], 0.0) +
                       jnp.where(m1, w2_ref[...], 0.0),
                       axis=-1, keepdims=True)
        xt = jnp.dot(sel, x_ref[...], preferred_element_type=jnp.float32)
        g = jnp.dot(xt, wg_ref[0], preferred_element_type=jnp.float32)
        u = jnp.dot(xt, wu_ref[0], preferred_element_type=jnp.float32)
        h = g * jax.nn.sigmoid(g) * u
        y = jnp.dot(h, wd_ref[0], preferred_element_type=jnp.float32)
        out_ref[...] = y * wvec


def _run_ffn(tile_eid, tile_valid, x2d, w_gate, w_up, w_down, pos0r, pos1r,
             w1r, w2r):
    grid_spec = pltpu.PrefetchScalarGridSpec(
        num_scalar_prefetch=2,
        grid=(MAX_TILES,),
        in_specs=[
            pl.BlockSpec((T, D), lambda i, eid, val: (0, 0)),
            pl.BlockSpec((1, D, F), lambda i, eid, val: (eid[i], 0, 0)),
            pl.BlockSpec((1, D, F), lambda i, eid, val: (eid[i], 0, 0)),
            pl.BlockSpec((1, F, D), lambda i, eid, val: (eid[i], 0, 0)),
            pl.BlockSpec((1, T), lambda i, eid, val: (0, 0)),
            pl.BlockSpec((1, T), lambda i, eid, val: (0, 0)),
            pl.BlockSpec((1, T), lambda i, eid, val: (0, 0)),
            pl.BlockSpec((1, T), lambda i, eid, val: (0, 0)),
        ],
        out_specs=pl.BlockSpec((TM, D), lambda i, eid, val: (i, 0)),
    )
    return pl.pallas_call(
        _ffn_body,
        grid_spec=grid_spec,
        out_shape=jax.ShapeDtypeStruct((R, D), jnp.float32),
    )(tile_eid, tile_valid, x2d, w_gate, w_up, w_down, pos0r, pos1r, w1r,
      w2r)


_C_PER_W = T // NW
_C_VCH = D // 16


def kernel(x, Wg, W_gate, W_up, W_down):
    x2d = x.reshape(T, D)
    cnt_f, aux, w1, w2, pos0, pos1, eid2, val2 = _run_router(x2d, Wg)

    sc_combine = _sc_kernels()
    ys = _run_ffn(eid2.reshape(-1), val2.reshape(-1), x2d, W_gate, W_up,
                  W_down, pos0.reshape(1, T), pos1.reshape(1, T),
                  w1.reshape(1, T), w2.reshape(1, T))
    out2d = sc_combine(ys, pos0.reshape(-1), pos1.reshape(-1))

    return out2d.reshape(1, T, D), aux[0, 0], cnt_f


# speedup vs baseline: 2.0598x; 1.0002x over previous
import functools

import jax
import jax.numpy as jnp
from jax import lax
from jax.experimental import pallas as pl
from jax.experimental.pallas import tpu as pltpu
from jax.experimental.pallas import tpu_sc as plsc

T = 2048
D = 768
F = 2048
NE = 64
K = 2
TM = 128
MAX_TILES = (T * K) // TM + NE
R = MAX_TILES * TM

NC = 2
NS = 16
NW = NC * NS


def _router_body(x_ref, wg_ref, cnt_ref, aux_ref, w1_ref, w2_ref, p1_ref,
                 p2_ref, eid_ref, val_ref):
    x = x_ref[...]
    logits = jnp.dot(x, wg_ref[...], preferred_element_type=jnp.float32)
    m = jnp.max(logits, axis=-1, keepdims=True)
    p = jnp.exp(logits - m)
    p = p / jnp.sum(p, axis=-1, keepdims=True)
    iota = lax.broadcasted_iota(jnp.int32, (T, NE), 1)
    p1 = jnp.max(p, axis=-1, keepdims=True)
    e1 = jnp.min(jnp.where(p == p1, iota, NE), axis=-1, keepdims=True)
    pm = jnp.where(iota == e1, -1e30, p)
    p2 = jnp.max(pm, axis=-1, keepdims=True)
    e2 = jnp.min(jnp.where(pm == p2, iota, NE), axis=-1, keepdims=True)
    s = p1 + p2
    w1_ref[...] = p1 / s
    w2_ref[...] = p2 / s
    hit1 = (iota == e1).astype(jnp.float32)
    hit2 = (iota == e2).astype(jnp.float32)
    hits = hit1 + hit2
    cnt = jnp.sum(hits, axis=0)
    cnt_ref[...] = cnt
    pmean = jnp.mean(p, axis=0)
    f = cnt / jnp.sum(cnt)
    aux_ref[...] = (NE * jnp.sum(f * pmean))[None, None]

    cnt_row = cnt[None, :]
    tiles_pe = (cnt_row.astype(jnp.int32) + TM - 1) // TM
    i64r = lax.broadcasted_iota(jnp.int32, (NE, NE), 0)
    i64c = lax.broadcasted_iota(jnp.int32, (NE, NE), 1)
    m_le = (i64r <= i64c).astype(jnp.float32)
    tile_cum = jnp.dot(tiles_pe.astype(jnp.float32), m_le,
                       preferred_element_type=jnp.float32).astype(jnp.int32)
    padded_start = ((tile_cum - tiles_pe) * TM).astype(jnp.float32)

    tb = 256
    ibr = lax.broadcasted_iota(jnp.int32, (tb, tb), 0)
    ibc = lax.broadcasted_iota(jnp.int32, (tb, tb), 1)
    l_strict = (ibc < ibr).astype(jnp.float32)
    off = jnp.zeros((1, NE), jnp.float32)
    for b in range(T // tb):
        sl = slice(b * tb, (b + 1) * tb)
        hb = hits[sl]
        cumex_b = jnp.dot(l_strict, hb,
                          preferred_element_type=jnp.float32) + off
        off = off + jnp.sum(hb, axis=0, keepdims=True)
        tgt_b = padded_start + cumex_b
        p1_ref[sl, :] = jnp.sum(hit1[sl] * tgt_b, axis=-1,
                                keepdims=True).astype(jnp.int32)
        p2_ref[sl, :] = jnp.sum(hit2[sl] * tgt_b, axis=-1,
                                keepdims=True).astype(jnp.int32)

    ti = lax.broadcasted_iota(jnp.int32, (MAX_TILES, NE), 0)
    eid = jnp.sum((tile_cum <= ti).astype(jnp.int32), axis=-1,
                  keepdims=True)
    eid_ref[...] = jnp.minimum(eid, NE - 1)
    total = jnp.max(tile_cum)
    vi = lax.broadcasted_iota(jnp.int32, (MAX_TILES, 1), 0)
    val_ref[...] = (vi < total).astype(jnp.int32)


def _run_router(x2d, wg):
    return pl.pallas_call(
        _router_body,
        out_shape=(
            jax.ShapeDtypeStruct((NE,), jnp.float32),
            jax.ShapeDtypeStruct((1, 1), jnp.float32),
            jax.ShapeDtypeStruct((T, 1), jnp.float32),
            jax.ShapeDtypeStruct((T, 1), jnp.float32),
            jax.ShapeDtypeStruct((T, 1), jnp.int32),
            jax.ShapeDtypeStruct((T, 1), jnp.int32),
            jax.ShapeDtypeStruct((MAX_TILES, 1), jnp.int32),
            jax.ShapeDtypeStruct((MAX_TILES, 1), jnp.int32),
        ),
    )(x2d, wg)


@functools.lru_cache(maxsize=None)
def _sc_kernels():
    mesh = plsc.VectorSubcoreMesh(core_axis_name="c", subcore_axis_name="s")

    @functools.partial(
        pl.kernel,
        out_type=jax.ShapeDtypeStruct((T, D), jnp.float32),
        mesh=mesh,
        scratch_types=[
            pltpu.VMEM((_C_PER_W,), jnp.int32),
            pltpu.VMEM((_C_PER_W,), jnp.int32),
            pltpu.VMEM((_C_PER_W, D), jnp.float32),
            pltpu.VMEM((_C_PER_W, D), jnp.float32),
            pltpu.SemaphoreType.DMA,
        ],
    )
    def _sc_combine(ys_hbm, pos0_hbm, pos1_hbm, out_hbm, i0_v, i1_v, b0_v,
                    b1_v, sem):
        wid = lax.axis_index("s") * NC + lax.axis_index("c")
        base = wid * _C_PER_W
        pltpu.sync_copy(pos0_hbm.at[pl.ds(base, _C_PER_W)], i0_v)
        pltpu.sync_copy(pos1_hbm.at[pl.ds(base, _C_PER_W)], i1_v)
        pltpu.async_copy(ys_hbm.at[i0_v], b0_v, sem).wait()
        pltpu.async_copy(ys_hbm.at[i1_v], b1_v, sem).wait()

        def _row(r, carry):
            for c in range(_C_VCH):
                sl = pl.ds(c * 16, 16)
                b0_v[r, sl] = b0_v[r, sl] + b1_v[r, sl]
            return carry

        lax.fori_loop(0, _C_PER_W, _row, 0)
        pltpu.sync_copy(b0_v, out_hbm.at[pl.ds(base, _C_PER_W)])

    return _sc_combine


def _ffn_body(eid_ref, val_ref, x_ref, wg_ref, wu_ref, wd_ref, pos0_ref,
              pos1_ref, w1_ref, w2_ref, out_ref):
    i = pl.program_id(0)

    @pl.when(val_ref[i] == 1)
    def _compute():
        rid = lax.broadcasted_iota(jnp.int32, (TM, T), 0) + i * TM
        m0 = pos0_ref[...] == rid
        m1 = pos1_ref[...] == rid
        sel = (m0 | m1).astype(jnp.float32)
        wvec = jnp.sum(jnp.where(m0, w1_ref[...], 0.0) +
                       jnp.where(m1, w2_ref[...], 0.0),
                       axis=-1, keepdims=True)
        xt = jnp.dot(sel, x_ref[...], preferred_element_type=jnp.float32)
        xtb = xt.astype(jnp.bfloat16)
        g = jnp.dot(xtb, wg_ref[0].astype(jnp.bfloat16),
                    preferred_element_type=jnp.float32)
        u = jnp.dot(xtb, wu_ref[0].astype(jnp.bfloat16),
                    preferred_element_type=jnp.float32)
        h = g * jax.nn.sigmoid(g) * u
        y = jnp.dot(h.astype(jnp.bfloat16), wd_ref[0].astype(jnp.bfloat16),
                    preferred_element_type=jnp.float32)
        out_ref[...] = y * wvec


def _run_ffn(tile_eid, tile_valid, x2d, w_gate, w_up, w_down, pos0r, pos1r,
             w1r, w2r):
    grid_spec = pltpu.PrefetchScalarGridSpec(
        num_scalar_prefetch=2,
        grid=(MAX_TILES,),
        in_specs=[
            pl.BlockSpec((T, D), lambda i, eid, val: (0, 0)),
            pl.BlockSpec((1, D, F), lambda i, eid, val: (eid[i], 0, 0)),
            pl.BlockSpec((1, D, F), lambda i, eid, val: (eid[i], 0, 0)),
            pl.BlockSpec((1, F, D), lambda i, eid, val: (eid[i], 0, 0)),
            pl.BlockSpec((1, T), lambda i, eid, val: (0, 0)),
            pl.BlockSpec((1, T), lambda i, eid, val: (0, 0)),
            pl.BlockSpec((1, T), lambda i, eid, val: (0, 0)),
            pl.BlockSpec((1, T), lambda i, eid, val: (0, 0)),
        ],
        out_specs=pl.BlockSpec((TM, D), lambda i, eid, val: (i, 0)),
    )
    return pl.pallas_call(
        _ffn_body,
        grid_spec=grid_spec,
        out_shape=jax.ShapeDtypeStruct((R, D), jnp.float32),
    )(tile_eid, tile_valid, x2d, w_gate, w_up, w_down, pos0r, pos1r, w1r,
      w2r)


_C_PER_W = T // NW
_C_VCH = D // 16


def kernel(x, Wg, W_gate, W_up, W_down):
    x2d = x.reshape(T, D)
    cnt_f, aux, w1, w2, pos0, pos1, eid2, val2 = _run_router(x2d, Wg)

    sc_combine = _sc_kernels()
    ys = _run_ffn(eid2.reshape(-1), val2.reshape(-1), x2d, W_gate, W_up,
                  W_down, pos0.reshape(1, T), pos1.reshape(1, T),
                  w1.reshape(1, T), w2.reshape(1, T))
    out2d = sc_combine(ys, pos0.reshape(-1), pos1.reshape(-1))

    return out2d.reshape(1, T, D), aux[0, 0], cnt_f
